# baseline (speedup 1.0000x reference)
import functools

import jax
import jax.numpy as jnp
from jax import lax
from jax.experimental import pallas as pl
from jax.experimental.pallas import tpu as pltpu
from jax.experimental.pallas import tpu_sc as plsc

NUM_EMB = 1000000
D = 32
BATCH = 4096
HIST = 50
TOTAL = BATCH * HIST

NC = 2
NS = 16
NW = NC * NS
PER_W = TOTAL // NW
CHUNK = 128
NCHUNK = PER_W // CHUNK
K = 5
NGROUP = NCHUNK // K
GROUP_ROWS = K * CHUNK

@functools.cache
def _build_kernel():
    mesh = plsc.VectorSubcoreMesh(
        core_axis_name="c", subcore_axis_name="s", num_cores=NC, num_subcores=NS
    )

    @functools.partial(
        pl.kernel,
        mesh=mesh,
        compiler_params=pltpu.CompilerParams(use_tc_tiling_on_sc=False),
        out_type=jax.ShapeDtypeStruct((TOTAL, D), jnp.float32),
        scratch_types=[
            pltpu.VMEM((NCHUNK, CHUNK), jnp.int32),
            pltpu.VMEM((GROUP_ROWS, D), jnp.float32),
            pltpu.SemaphoreType.DMA,
        ],
    )
    def _emb_lookup(idx_hbm, w_hbm, out_hbm, idx_v, rows_v, gsem):
        wid = lax.axis_index("s") * NC + lax.axis_index("c")
        base = wid * PER_W

        pltpu.sync_copy(idx_hbm.at[wid], idx_v)

        def group_body(g, carry):
            for j in range(K):
                pltpu.async_copy(
                    w_hbm.at[idx_v.at[g * K + j]],
                    rows_v.at[pl.ds(j * CHUNK, CHUNK)],
                    gsem,
                )
            pltpu.make_async_copy(
                w_hbm.at[pl.ds(0, GROUP_ROWS)], rows_v, gsem
            ).wait()
            pltpu.sync_copy(
                rows_v, out_hbm.at[pl.ds(base + g * GROUP_ROWS, GROUP_ROWS)]
            )
            return carry

        lax.fori_loop(0, NGROUP, group_body, 0)

    return _emb_lookup


def kernel(input, weight):
    idx = input.astype(jnp.int32).reshape(NW, NCHUNK, CHUNK)
    out = _build_kernel()(idx, weight)
    return out.reshape(BATCH, HIST, D)


# speedup vs baseline: 1.0082x; 1.0082x over previous
import functools

import jax
import jax.numpy as jnp
from jax import lax
from jax.experimental import pallas as pl
from jax.experimental.pallas import tpu as pltpu
from jax.experimental.pallas import tpu_sc as plsc

NUM_EMB = 1000000
D = 32
BATCH = 4096
HIST = 50
TOTAL = BATCH * HIST

NC = 2
NS = 16
NW = NC * NS
PER_W = TOTAL // NW
CHUNK = 128
NCHUNK = PER_W // CHUNK
K = 10
NGROUP = NCHUNK // K
GROUP_ROWS = K * CHUNK

@functools.cache
def _build_kernel():
    mesh = plsc.VectorSubcoreMesh(
        core_axis_name="c", subcore_axis_name="s", num_cores=NC, num_subcores=NS
    )

    @functools.partial(
        pl.kernel,
        mesh=mesh,
        compiler_params=pltpu.CompilerParams(use_tc_tiling_on_sc=False),
        out_type=jax.ShapeDtypeStruct((TOTAL, D), jnp.float32),
        scratch_types=[
            pltpu.VMEM((NCHUNK, CHUNK), jnp.int32),
            pltpu.VMEM((2, GROUP_ROWS, D), jnp.float32),
            pltpu.SemaphoreType.DMA,
            pltpu.SemaphoreType.DMA,
            pltpu.SemaphoreType.DMA,
            pltpu.SemaphoreType.DMA,
        ],
    )
    def _emb_lookup(
        idx_hbm, w_hbm, out_hbm, idx_v, rows_v, gsem0, gsem1, wsem0, wsem1
    ):
        wid = lax.axis_index("s") * NC + lax.axis_index("c")
        base = wid * PER_W
        gsems = (gsem0, gsem1)
        wsems = (wsem0, wsem1)

        pltpu.sync_copy(idx_hbm.at[wid], idx_v)

        def fire_gathers(g, slot):
            for j in range(K):
                pltpu.async_copy(
                    w_hbm.at[idx_v.at[g * K + j]],
                    rows_v.at[slot, pl.ds(j * CHUNK, CHUNK)],
                    gsems[slot],
                )

        def drain_gathers(slot):
            pltpu.make_async_copy(
                w_hbm.at[pl.ds(0, GROUP_ROWS)], rows_v.at[slot], gsems[slot]
            ).wait()

        def fire_write(g, slot):
            pltpu.async_copy(
                rows_v.at[slot],
                out_hbm.at[pl.ds(base + g * GROUP_ROWS, GROUP_ROWS)],
                wsems[slot],
            )

        def drain_write(slot):
            pltpu.make_async_copy(
                rows_v.at[slot],
                out_hbm.at[pl.ds(base, GROUP_ROWS)],
                wsems[slot],
            ).wait()

        def per_slot(g, fn):
            pl.when(g % 2 == 0)(lambda: fn(0))
            pl.when(g % 2 == 1)(lambda: fn(1))

        fire_gathers(0, 0)

        def group_body(g, carry):
            pl.when(g >= 2)(lambda: per_slot(g, drain_write))
            per_slot(g, lambda s: fire_gathers(g, s))
            per_slot(g - 1, drain_gathers)
            per_slot(g - 1, lambda s: fire_write(g - 1, s))
            return carry

        lax.fori_loop(1, NGROUP, group_body, 0)

        last = NGROUP - 1
        per_slot(last, drain_gathers)
        per_slot(last, lambda s: fire_write(last, s))
        per_slot(last - 1, drain_write)
        per_slot(last, drain_write)

    return _emb_lookup


def kernel(input, weight):
    idx = input.astype(jnp.int32).reshape(NW, NCHUNK, CHUNK)
    out = _build_kernel()(idx, weight)
    return out.reshape(BATCH, HIST, D)
